# baseline (speedup 1.0000x reference)
import dataclasses
import functools

import jax
import jax.numpy as jnp
from jax import lax
from jax.experimental import pallas as pl
from jax.experimental.pallas import tpu as pltpu
from jax.experimental.pallas import tpu_sc as plsc

N = 10000
E = 320000
C_IN = 128
C_HID = 128
C_OUT = 64
HEADS = 4
CW = 32

NC = 2
NS = 16
NW = NC * NS

B = 128
NBUF = 8
PF = 4
J0 = 160
TOTB = NS * J0
E_PAD = TOTB * B
DEG_J = TOTB // NW
ACC_ROWS = 10240
ROWS_PER_TILE_Z = ACC_ROWS // NS
FLUSH_SMALL = 624
FLUSH_LAST = N - (NS - 1) * FLUSH_SMALL
DUMP = N


@functools.cache
def _mesh():
    return plsc.VectorSubcoreMesh(
        core_axis_name="c", subcore_axis_name="s", num_cores=NC, num_subcores=NS
    )


@functools.cache
def _mesh1():
    return plsc.VectorSubcoreMesh(
        core_axis_name="c", subcore_axis_name="s", num_cores=1, num_subcores=NS
    )


_SC_PARAMS = pltpu.CompilerParams(use_tc_tiling_on_sc=False)
_SC_PARAMS_NOLAYOUT = dataclasses.replace(_SC_PARAMS, needs_layout_passes=False)


def _zero_buf(buf):
    rows, cols = buf.shape

    @pl.loop(0, rows)
    def _(i):
        @pl.loop(0, cols, step=16)
        def _(k):
            buf[i, pl.ds(k, 16)] = jnp.zeros((16,), jnp.float32)


def _fill_ones(buf):
    rows, cols = buf.shape

    @pl.loop(0, rows)
    def _(i):
        @pl.loop(0, cols, step=16)
        def _(k):
            buf[i, pl.ds(k, 16)] = jnp.ones((16,), jnp.float32)


def _zero_acc(acc, zbuf, sid):
    nz = ROWS_PER_TILE_Z // B

    @pl.loop(0, nz)
    def _(r):
        pltpu.sync_copy(zbuf, acc.at[pl.ds(sid * ROWS_PER_TILE_Z + r * B, B)])


def _flush(acc, out_hbm, cid, sid):
    start = pl.multiple_of(sid * FLUSH_SMALL, 8)

    @pl.when(sid < NS - 1)
    def _():
        pltpu.sync_copy(
            acc.at[pl.ds(start, FLUSH_SMALL)],
            out_hbm.at[cid].at[pl.ds(start, FLUSH_SMALL)],
        )

    @pl.when(sid == NS - 1)
    def _():
        base = (NS - 1) * FLUSH_SMALL
        pltpu.sync_copy(
            acc.at[pl.ds(base, FLUSH_LAST)],
            out_hbm.at[cid].at[pl.ds(base, FLUSH_LAST)],
        )


@functools.cache
def _make_deg():
    return functools.partial(
        pl.kernel,
        out_type=jax.ShapeDtypeStruct((NW, ACC_ROWS), jnp.float32),
        mesh=_mesh(),
        scratch_types=[
            pltpu.VMEM((DEG_J, B), jnp.int32),
            pltpu.VMEM((ACC_ROWS,), jnp.float32),
        ],
        compiler_params=_SC_PARAMS_NOLAYOUT,
    )(_deg_body)


def _deg_body(dst_hbm, out_hbm, dst_v, hist):
    cid = lax.axis_index("c")
    sid = lax.axis_index("s")
    wid = cid * NS + sid

    pltpu.sync_copy(dst_hbm.at[wid], dst_v)

    @pl.loop(0, ACC_ROWS, step=16)
    def _(i):
        hist[pl.ds(i, 16)] = jnp.zeros((16,), jnp.float32)

    ones = jnp.ones((16,), jnp.float32)

    @pl.loop(0, DEG_J)
    def _(j):
        @pl.loop(0, B, step=16)
        def _(k):
            idx = dst_v[j, pl.ds(k, 16)]
            plsc.addupdate_scatter(hist, [idx], ones)

    pltpu.sync_copy(hist, out_hbm.at[wid])


@functools.cache
def _make_spmm(n_tab):
    @functools.partial(
        pl.kernel,
        out_type=[jax.ShapeDtypeStruct((1, N, CW), jnp.float32)] * n_tab,
        mesh=_mesh1(),
        scratch_types=(
            [
                pltpu.VMEM((J0, B), jnp.int32),
                pltpu.VMEM((J0, B), jnp.int32),
            ]
            + [pltpu.VMEM((B, CW), jnp.float32)] * NBUF
            + [pltpu.VMEM_SHARED((ACC_ROWS, CW), jnp.float32)]
            + [pltpu.SemaphoreType.DMA] * (2 * NBUF)
        ),
        compiler_params=_SC_PARAMS,
    )
    def spmm(src_hbm, dst_hbm, *rest):
        tabs = rest[:n_tab]
        outs = rest[n_tab : 2 * n_tab]
        sc = rest[2 * n_tab :]
        src_v, dst_v = sc[0], sc[1]
        bufs = sc[2 : 2 + NBUF]
        acc = sc[2 + NBUF]
        sem_g = sc[3 + NBUF : 3 + 2 * NBUF]
        sem_s = sc[3 + 2 * NBUF :]

        cid = lax.axis_index("c")
        sid = lax.axis_index("s")

        pltpu.sync_copy(src_hbm.at[sid], src_v)
        pltpu.sync_copy(dst_hbm.at[sid], dst_v)

        def gather(tab, m, b):
            pltpu.async_copy(tab.at[src_v.at[m]], bufs[b], sem_g[b])

        def wait_gather(tab, j, b):
            pltpu.make_async_copy(tab.at[src_v.at[j]], bufs[b], sem_g[b]).wait()

        def scatter(j, b):
            pltpu.async_copy(bufs[b], acc.at[dst_v.at[j]], sem_s[b], add=True)

        def wait_scatter(j, b):
            pltpu.make_async_copy(bufs[b], acc.at[dst_v.at[j]], sem_s[b]).wait()

        def deep(tab):
            def slot(j, b, wait_s, do_gather):
                wait_gather(tab, j, b)
                scatter(j, b)
                bn = (b + PF) % NBUF
                if wait_s:
                    wait_scatter(j, bn)
                if do_gather:
                    gather(tab, j + PF, bn)

            for m in range(PF):
                gather(tab, m, m)

            for b in range(NBUF):
                slot(b, b, wait_s=(b >= PF), do_gather=True)

            @pl.loop(1, J0 // NBUF - 1)
            def _(g):
                jg = g * NBUF
                for b in range(NBUF):
                    slot(jg + b, b, wait_s=True, do_gather=True)

            for b in range(NBUF):
                j = (J0 - NBUF) + b
                slot(j, b, wait_s=True, do_gather=(j + PF < J0))

            for j in range(J0 - PF, J0):
                wait_scatter(j, j % NBUF)

        for t in range(n_tab):
            tab = tabs[t]

            _zero_buf(bufs[0])
            _zero_acc(acc, bufs[0], sid)
            plsc.subcore_barrier()

            deep(tab)

            plsc.subcore_barrier()
            _flush(acc, outs[t], cid, sid)
            plsc.subcore_barrier()

    return spmm


def _dinv_from_degp(degp):
    deg = jnp.sum(degp, axis=1, keepdims=True) + 1.0
    return lax.rsqrt(deg)


NT1 = C_HID // CW
NT2 = HEADS * C_OUT // CW


def _combine(pref):
    s = pref[0]
    for c in range(1, pref.shape[0]):
        s = s + pref[c]
    return s


def _tc1_body(x_ref, w1_ref, degp_ref, *xs_refs):
    dinv = _dinv_from_degp(degp_ref[...])
    xw = jnp.dot(x_ref[...], w1_ref[...], preferred_element_type=jnp.float32)
    xs = xw * dinv
    for k, xs_ref in enumerate(xs_refs):
        xs_ref[...] = xs[:, k * CW : (k + 1) * CW]


def _tc2_body(*refs):
    s1_refs = refs[:NT1]
    xs_refs = refs[NT1 : 2 * NT1]
    degp_ref, b1_ref, wof_ref = refs[2 * NT1 : 2 * NT1 + 3]
    hs_refs = refs[2 * NT1 + 3 :]
    dinv = _dinv_from_degp(degp_ref[...])
    hs = None
    for k in range(NT1):
        hk = jnp.maximum(
            (_combine(s1_refs[k]) + xs_refs[k][...]) * dinv
            + b1_ref[:, k * CW : (k + 1) * CW],
            0.0,
        )
        part = jnp.dot(
            hk,
            wof_ref[k * CW : (k + 1) * CW, :],
            preferred_element_type=jnp.float32,
        )
        hs = part if hs is None else hs + part
    hs = hs * dinv
    for k, hs_ref in enumerate(hs_refs):
        hs_ref[...] = hs[:, k * CW : (k + 1) * CW]


def _tc3_body(*refs):
    s2_refs = refs[:NT2]
    hs_refs = refs[NT2 : 2 * NT2]
    degp_ref, bof_ref, o_ref = refs[2 * NT2 :]
    dinv = _dinv_from_degp(degp_ref[...])
    for k in range(NT2):
        ok = (_combine(s2_refs[k]) + hs_refs[k][...]) * dinv
        o_ref[:, k * CW : (k + 1) * CW] = ok + bof_ref[:, k * CW : (k + 1) * CW]


def _row_block(shape, rb, row_axis):
    blk = list(shape)
    blk[row_axis] = rb
    nd = len(shape)

    def idx(i):
        return tuple(i if d == row_axis else 0 for d in range(nd))

    return pl.BlockSpec(tuple(blk), idx)


def kernel(x, edge_index, W1, b1, Wo, bo):
    src = edge_index[0].astype(jnp.int32)
    dst = edge_index[1].astype(jnp.int32)

    pad = E_PAD - E
    src_f = jnp.concatenate([src, jnp.zeros((pad,), jnp.int32)])
    dst_f = jnp.concatenate([dst, jnp.full((pad,), DUMP, jnp.int32)])
    src0 = src_f.reshape(NS, J0, B)
    dst0 = dst_f.reshape(NS, J0, B)
    dst_deg = dst_f.reshape(NW, DEG_J, B)

    wof = Wo.transpose(1, 0, 2).reshape(C_HID, HEADS * C_OUT)
    bof = bo.reshape(1, HEADS * C_OUT)
    b1r = b1.reshape(1, C_HID)

    degp = _make_deg()(dst_deg).T

    rb = 2000
    grid = (N // rb,)
    f32 = jnp.float32
    degp_spec = _row_block((ACC_ROWS, NW), rb, 0)

    xs = pl.pallas_call(
        _tc1_body,
        grid=grid,
        in_specs=[
            _row_block((N, C_IN), rb, 0),
            pl.BlockSpec((C_IN, C_HID), lambda i: (0, 0)),
            degp_spec,
        ],
        out_specs=[_row_block((N, CW), rb, 0)] * NT1,
        out_shape=[jax.ShapeDtypeStruct((N, CW), f32)] * NT1,
    )(x, W1, degp)

    s1 = _make_spmm(NT1)(src0, dst0, *xs)

    hs = pl.pallas_call(
        _tc2_body,
        grid=grid,
        in_specs=[_row_block((1, N, CW), rb, 1)] * NT1
        + [_row_block((N, CW), rb, 0)] * NT1
        + [
            degp_spec,
            pl.BlockSpec((1, C_HID), lambda i: (0, 0)),
            pl.BlockSpec((C_HID, HEADS * C_OUT), lambda i: (0, 0)),
        ],
        out_specs=[_row_block((N, CW), rb, 0)] * NT2,
        out_shape=[jax.ShapeDtypeStruct((N, CW), f32)] * NT2,
    )(*s1, *xs, degp, b1r, wof)

    s2 = _make_spmm(NT2)(src0, dst0, *hs)

    out_flat = pl.pallas_call(
        _tc3_body,
        grid=grid,
        in_specs=[_row_block((1, N, CW), rb, 1)] * NT2
        + [_row_block((N, CW), rb, 0)] * NT2
        + [
            degp_spec,
            pl.BlockSpec((1, HEADS * C_OUT), lambda i: (0, 0)),
        ],
        out_specs=_row_block((N, HEADS * C_OUT), rb, 0),
        out_shape=jax.ShapeDtypeStruct((N, HEADS * C_OUT), f32),
    )(*s2, *hs, degp, bof)

    return out_flat.reshape(N, HEADS, C_OUT).transpose(1, 0, 2)


# speedup vs baseline: 1.3149x; 1.3149x over previous
import dataclasses
import functools

import jax
import jax.numpy as jnp
from jax import lax
from jax.experimental import pallas as pl
from jax.experimental.pallas import tpu as pltpu
from jax.experimental.pallas import tpu_sc as plsc

N = 10000
E = 320000
C_IN = 128
C_HID = 128
C_OUT = 64
HEADS = 4
CW = 32

NC = 2
NS = 16
NW = NC * NS

B = 128
NBUF = 8
PF = 4
J0 = 144
J1 = 16
TOTB = NS * (J0 + J1)
BASE1 = NS * J0
E_PAD = TOTB * B
DEG_J = TOTB // NW
ACC_ROWS = 10240
ROWS_PER_TILE_Z = ACC_ROWS // NS
FLUSH_SMALL = 624
FLUSH_LAST = N - (NS - 1) * FLUSH_SMALL
DUMP = N


@functools.cache
def _mesh():
    return plsc.VectorSubcoreMesh(
        core_axis_name="c", subcore_axis_name="s", num_cores=NC, num_subcores=NS
    )


@functools.cache
def _mesh1():
    return plsc.VectorSubcoreMesh(
        core_axis_name="c", subcore_axis_name="s", num_cores=1, num_subcores=NS
    )


_SC_PARAMS = pltpu.CompilerParams(use_tc_tiling_on_sc=False)
_SC_PARAMS_NOLAYOUT = dataclasses.replace(_SC_PARAMS, needs_layout_passes=False)


def _zero_buf(buf):
    rows, cols = buf.shape

    @pl.loop(0, rows)
    def _(i):
        @pl.loop(0, cols, step=16)
        def _(k):
            buf[i, pl.ds(k, 16)] = jnp.zeros((16,), jnp.float32)


def _fill_ones(buf):
    rows, cols = buf.shape

    @pl.loop(0, rows)
    def _(i):
        @pl.loop(0, cols, step=16)
        def _(k):
            buf[i, pl.ds(k, 16)] = jnp.ones((16,), jnp.float32)


def _zero_acc(acc, zbuf, sid):
    nz = ROWS_PER_TILE_Z // B

    @pl.loop(0, nz)
    def _(r):
        pltpu.sync_copy(zbuf, acc.at[pl.ds(sid * ROWS_PER_TILE_Z + r * B, B)])


def _flush(acc, out_hbm, cid, sid):
    start = pl.multiple_of(sid * FLUSH_SMALL, 8)

    @pl.when(sid < NS - 1)
    def _():
        pltpu.sync_copy(
            acc.at[pl.ds(start, FLUSH_SMALL)],
            out_hbm.at[cid].at[pl.ds(start, FLUSH_SMALL)],
        )

    @pl.when(sid == NS - 1)
    def _():
        base = (NS - 1) * FLUSH_SMALL
        pltpu.sync_copy(
            acc.at[pl.ds(base, FLUSH_LAST)],
            out_hbm.at[cid].at[pl.ds(base, FLUSH_LAST)],
        )


@functools.cache
def _make_deg():
    return functools.partial(
        pl.kernel,
        out_type=jax.ShapeDtypeStruct((NW, ACC_ROWS), jnp.float32),
        mesh=_mesh(),
        scratch_types=[
            pltpu.VMEM((DEG_J, B), jnp.int32),
            pltpu.VMEM((ACC_ROWS,), jnp.float32),
        ],
        compiler_params=_SC_PARAMS_NOLAYOUT,
    )(_deg_body)


def _deg_body(dst_hbm, out_hbm, dst_v, hist):
    cid = lax.axis_index("c")
    sid = lax.axis_index("s")
    wid = cid * NS + sid

    pltpu.sync_copy(dst_hbm.at[wid], dst_v)

    @pl.loop(0, ACC_ROWS, step=16)
    def _(i):
        hist[pl.ds(i, 16)] = jnp.zeros((16,), jnp.float32)

    ones = jnp.ones((16,), jnp.float32)

    @pl.loop(0, DEG_J)
    def _(j):
        @pl.loop(0, B, step=16)
        def _(k):
            idx = dst_v[j, pl.ds(k, 16)]
            plsc.addupdate_scatter(hist, [idx], ones)

    pltpu.sync_copy(hist, out_hbm.at[wid])


@functools.cache
def _make_spmm(n_tab):
    @functools.partial(
        pl.kernel,
        out_type=[jax.ShapeDtypeStruct((NC, N, CW), jnp.float32)] * n_tab,
        mesh=_mesh(),
        scratch_types=(
            [
                pltpu.VMEM((J0, B), jnp.int32),
                pltpu.VMEM((J0, B), jnp.int32),
            ]
            + [pltpu.VMEM((B, CW), jnp.float32)] * NBUF
            + [pltpu.VMEM_SHARED((ACC_ROWS, CW), jnp.float32)]
            + [pltpu.SemaphoreType.DMA] * (2 * NBUF)
        ),
        compiler_params=_SC_PARAMS,
    )
    def spmm(src0_hbm, dst0_hbm, src1_hbm, dst1_hbm, *rest):
        tabs = rest[:n_tab]
        outs = rest[n_tab : 2 * n_tab]
        sc = rest[2 * n_tab :]
        src_v, dst_v = sc[0], sc[1]
        bufs = sc[2 : 2 + NBUF]
        acc = sc[2 + NBUF]
        sem_g = sc[3 + NBUF : 3 + 2 * NBUF]
        sem_s = sc[3 + 2 * NBUF :]

        cid = lax.axis_index("c")
        sid = lax.axis_index("s")

        @pl.when(cid == 0)
        def _():
            pltpu.sync_copy(src0_hbm.at[sid], src_v)
            pltpu.sync_copy(dst0_hbm.at[sid], dst_v)

        @pl.when(cid == 1)
        def _():
            pltpu.sync_copy(src1_hbm.at[sid], src_v.at[pl.ds(0, J1)])
            pltpu.sync_copy(dst1_hbm.at[sid], dst_v.at[pl.ds(0, J1)])

        def gather(tab, m, b):
            pltpu.async_copy(tab.at[src_v.at[m]], bufs[b], sem_g[b])

        def wait_gather(tab, j, b):
            pltpu.make_async_copy(tab.at[src_v.at[j]], bufs[b], sem_g[b]).wait()

        def scatter(j, b):
            pltpu.async_copy(bufs[b], acc.at[dst_v.at[j]], sem_s[b], add=True)

        def wait_scatter(j, b):
            pltpu.make_async_copy(bufs[b], acc.at[dst_v.at[j]], sem_s[b]).wait()

        def deep(tab):
            def slot(j, b, wait_s, do_gather):
                wait_gather(tab, j, b)
                scatter(j, b)
                bn = (b + PF) % NBUF
                if wait_s:
                    wait_scatter(j, bn)
                if do_gather:
                    gather(tab, j + PF, bn)

            for m in range(PF):
                gather(tab, m, m)

            for b in range(NBUF):
                slot(b, b, wait_s=(b >= PF), do_gather=True)

            @pl.loop(1, J0 // NBUF - 1)
            def _(g):
                jg = g * NBUF
                for b in range(NBUF):
                    slot(jg + b, b, wait_s=True, do_gather=True)

            for b in range(NBUF):
                j = (J0 - NBUF) + b
                slot(j, b, wait_s=True, do_gather=(j + PF < J0))

            for j in range(J0 - PF, J0):
                wait_scatter(j, j % NBUF)

        def shallow(tab):
            def sslot(j, b, do_issue):
                wait_gather(tab, j, b)
                pltpu.sync_copy(bufs[b], acc.at[dst_v.at[j]], add=True)
                if do_issue:
                    gather(tab, j + 2, b)

            gather(tab, 0, 0)
            gather(tab, 1, 1)

            @pl.loop(0, J1 // 2 - 1)
            def _(g):
                sslot(2 * g, 0, True)
                sslot(2 * g + 1, 1, True)

            sslot(J1 - 2, 0, False)
            sslot(J1 - 1, 1, False)

        for t in range(n_tab):
            tab = tabs[t]

            _zero_buf(bufs[0])
            _zero_acc(acc, bufs[0], sid)
            plsc.subcore_barrier()

            @pl.when(cid == 0)
            def _():
                deep(tab)

            @pl.when(cid == 1)
            def _():
                shallow(tab)

            plsc.subcore_barrier()
            _flush(acc, outs[t], cid, sid)
            plsc.subcore_barrier()

    return spmm


def _dinv_from_degp(degp):
    deg = jnp.sum(degp, axis=1, keepdims=True) + 1.0
    return lax.rsqrt(deg)


NT1 = C_HID // CW
NT2 = HEADS * C_OUT // CW


def _combine(pref):
    s = pref[0]
    for c in range(1, pref.shape[0]):
        s = s + pref[c]
    return s


def _tc1_body(x_ref, w1_ref, degp_ref, *xs_refs):
    dinv = _dinv_from_degp(degp_ref[...])
    xw = jnp.dot(x_ref[...], w1_ref[...], preferred_element_type=jnp.float32)
    xs = xw * dinv
    for k, xs_ref in enumerate(xs_refs):
        xs_ref[...] = xs[:, k * CW : (k + 1) * CW]


def _tc2_body(*refs):
    s1_refs = refs[:NT1]
    xs_refs = refs[NT1 : 2 * NT1]
    degp_ref, b1_ref, wof_ref = refs[2 * NT1 : 2 * NT1 + 3]
    hs_refs = refs[2 * NT1 + 3 :]
    dinv = _dinv_from_degp(degp_ref[...])
    hs = None
    for k in range(NT1):
        hk = jnp.maximum(
            (_combine(s1_refs[k]) + xs_refs[k][...]) * dinv
            + b1_ref[:, k * CW : (k + 1) * CW],
            0.0,
        )
        part = jnp.dot(
            hk,
            wof_ref[k * CW : (k + 1) * CW, :],
            preferred_element_type=jnp.float32,
        )
        hs = part if hs is None else hs + part
    hs = hs * dinv
    for k, hs_ref in enumerate(hs_refs):
        hs_ref[...] = hs[:, k * CW : (k + 1) * CW]


def _tc3_body(*refs):
    s2_refs = refs[:NT2]
    hs_refs = refs[NT2 : 2 * NT2]
    degp_ref, bof_ref, o_ref = refs[2 * NT2 :]
    dinv = _dinv_from_degp(degp_ref[...])
    for k in range(NT2):
        ok = (_combine(s2_refs[k]) + hs_refs[k][...]) * dinv
        o_ref[:, k * CW : (k + 1) * CW] = ok + bof_ref[:, k * CW : (k + 1) * CW]


def _row_block(shape, rb, row_axis):
    blk = list(shape)
    blk[row_axis] = rb
    nd = len(shape)

    def idx(i):
        return tuple(i if d == row_axis else 0 for d in range(nd))

    return pl.BlockSpec(tuple(blk), idx)


def kernel(x, edge_index, W1, b1, Wo, bo):
    src = edge_index[0].astype(jnp.int32)
    dst = edge_index[1].astype(jnp.int32)

    pad = E_PAD - E
    src_f = jnp.concatenate([src, jnp.zeros((pad,), jnp.int32)])
    dst_f = jnp.concatenate([dst, jnp.full((pad,), DUMP, jnp.int32)])
    split = BASE1 * B
    src0 = src_f[:split].reshape(NS, J0, B)
    dst0 = dst_f[:split].reshape(NS, J0, B)
    src1 = src_f[split:].reshape(NS, J1, B)
    dst1 = dst_f[split:].reshape(NS, J1, B)
    dst_deg = dst_f.reshape(NW, DEG_J, B)

    wof = Wo.transpose(1, 0, 2).reshape(C_HID, HEADS * C_OUT)
    bof = bo.reshape(1, HEADS * C_OUT)
    b1r = b1.reshape(1, C_HID)

    degp = _make_deg()(dst_deg).T

    rb = 2000
    grid = (N // rb,)
    f32 = jnp.float32
    degp_spec = _row_block((ACC_ROWS, NW), rb, 0)

    xs = pl.pallas_call(
        _tc1_body,
        grid=grid,
        in_specs=[
            _row_block((N, C_IN), rb, 0),
            pl.BlockSpec((C_IN, C_HID), lambda i: (0, 0)),
            degp_spec,
        ],
        out_specs=[_row_block((N, CW), rb, 0)] * NT1,
        out_shape=[jax.ShapeDtypeStruct((N, CW), f32)] * NT1,
    )(x, W1, degp)

    s1 = _make_spmm(NT1)(src0, dst0, src1, dst1, *xs)

    hs = pl.pallas_call(
        _tc2_body,
        grid=grid,
        in_specs=[_row_block((NC, N, CW), rb, 1)] * NT1
        + [_row_block((N, CW), rb, 0)] * NT1
        + [
            degp_spec,
            pl.BlockSpec((1, C_HID), lambda i: (0, 0)),
            pl.BlockSpec((C_HID, HEADS * C_OUT), lambda i: (0, 0)),
        ],
        out_specs=[_row_block((N, CW), rb, 0)] * NT2,
        out_shape=[jax.ShapeDtypeStruct((N, CW), f32)] * NT2,
    )(*s1, *xs, degp, b1r, wof)

    s2 = _make_spmm(NT2)(src0, dst0, src1, dst1, *hs)

    out_flat = pl.pallas_call(
        _tc3_body,
        grid=grid,
        in_specs=[_row_block((NC, N, CW), rb, 1)] * NT2
        + [_row_block((N, CW), rb, 0)] * NT2
        + [
            degp_spec,
            pl.BlockSpec((1, HEADS * C_OUT), lambda i: (0, 0)),
        ],
        out_specs=_row_block((N, HEADS * C_OUT), rb, 0),
        out_shape=jax.ShapeDtypeStruct((N, HEADS * C_OUT), f32),
    )(*s2, *hs, degp, bof)

    return out_flat.reshape(N, HEADS, C_OUT).transpose(1, 0, 2)
